# baseline (speedup 1.0000x reference)
import jax
import jax.numpy as jnp
from jax import lax
from jax.experimental import pallas as pl
from jax.experimental.pallas import tpu as pltpu
from jax.experimental.pallas import tpu_sc as plsc

NUM_CORES = 2
NUM_SUBCORES = 16
LANES = 16
NW = NUM_CORES * NUM_SUBCORES

NUM_ROWS = 1000000
BATCH = 16384
LATENT = 16
B_PER_W = BATCH // NW
CHUNK = 128
NCHUNK = B_PER_W // CHUNK
SUB = CHUNK // LANES

TPP = -(-NUM_ROWS // 128)
TILES_PER_W16 = -(-TPP // 16)
PADCOLS = TPP * 128
DBLK_STRIDE = PADCOLS * 8
FLAT = 2 * DBLK_STRIDE


def _retile_body(utab_ref, itab_ref, t3u_ref, t3i_ref, sem):
    wid = lax.axis_index("s") * NUM_CORES + lax.axis_index("c")
    tbl = wid % 2
    w16 = wid // 2

    start = jnp.minimum(w16 * TILES_PER_W16, TPP - TILES_PER_W16)
    co = pl.multiple_of(start * 128, 128)
    width = TILES_PER_W16 * 128

    def run(src_ref, dst_ref):
        cps = []
        for dblk in range(2):
            cp = pltpu.make_async_copy(
                src_ref.at[pl.ds(dblk * 8, 8), pl.ds(co, width)],
                dst_ref.at[dblk].at[pl.ds(0, 8), pl.ds(co, width)], sem)
            cp.start()
            cps.append(cp)
        for cp in cps:
            cp.wait()

    @pl.when(tbl == 0)
    def _():
        run(utab_ref, t3u_ref)

    @pl.when(tbl == 1)
    def _():
        run(itab_ref, t3i_ref)


def _gather_body(users_ref, items_ref, uflat_ref, iflat_ref, out_ref,
                 idx_u, idx_i, gidx_u, gidx_i, buf_u, buf_i, out_v,
                 sem_u, sem_i):
    wid = lax.axis_index("s") * NUM_CORES + lax.axis_index("c")
    base = wid * B_PER_W

    pltpu.sync_copy(users_ref.at[pl.ds(base, B_PER_W)], idx_u)
    pltpu.sync_copy(items_ref.at[pl.ds(base, B_PER_W)], idx_i)

    def build(c, _):
        for t in range(SUB):
            sl = pl.ds(c * CHUNK + t * LANES, LANES)
            tsl = pl.ds(t * LANES, LANES)
            ru = idx_u[sl]
            ri = idx_i[sl]
            bu = ((ru >> 7) << 10) + (ru & 127)
            bi = ((ri >> 7) << 10) + (ri & 127)
            for d in range(LATENT):
                cd = (d // 8) * DBLK_STRIDE + (d % 8) * 128
                gidx_u[d, c, tsl] = bu + cd
                gidx_i[d, c, tsl] = bi + cd
        return 0

    lax.fori_loop(0, NCHUNK, build, 0)

    copies = []
    for d in range(LATENT):
        for c in range(NCHUNK):
            sl = pl.ds(c * CHUNK, CHUNK)
            cu = pltpu.make_async_copy(
                uflat_ref.at[gidx_u.at[d].at[c]], buf_u.at[d].at[sl], sem_u)
            ci = pltpu.make_async_copy(
                iflat_ref.at[gidx_i.at[d].at[c]], buf_i.at[d].at[sl], sem_i)
            cu.start()
            ci.start()
            copies.append(cu)
            copies.append(ci)
    for cp in copies:
        cp.wait()

    def group(g, _):
        sl = pl.ds(g * LANES, LANES)
        acc = jnp.zeros((LANES,), jnp.float32)
        for d in range(LATENT):
            acc = acc + buf_u[d, sl] * buf_i[d, sl]
        out_v[sl] = 1.0 / (1.0 + jnp.exp(-acc))
        return 0

    lax.fori_loop(0, B_PER_W // LANES, group, 0)

    pltpu.sync_copy(out_v, out_ref.at[pl.ds(base, B_PER_W)])


@jax.jit
def kernel(users, items, user_table, item_table):
    ut2 = user_table.T
    it2 = item_table.T
    mesh = plsc.VectorSubcoreMesh(
        core_axis_name="c", subcore_axis_name="s",
        num_cores=NUM_CORES, num_subcores=NUM_SUBCORES)

    retile = pl.kernel(
        _retile_body,
        out_type=(
            jax.ShapeDtypeStruct((2, 8, PADCOLS), jnp.float32),
            jax.ShapeDtypeStruct((2, 8, PADCOLS), jnp.float32),
        ),
        mesh=mesh,
        scratch_types=[
            pltpu.SemaphoreType.DMA,
        ],
    )
    t3u, t3i = retile(ut2, it2)
    uflat = t3u.reshape(FLAT)
    iflat = t3i.reshape(FLAT)

    gather = pl.kernel(
        _gather_body,
        out_type=jax.ShapeDtypeStruct((BATCH,), jnp.float32),
        mesh=mesh,
        scratch_types=[
            pltpu.VMEM((B_PER_W,), jnp.int32),
            pltpu.VMEM((B_PER_W,), jnp.int32),
            pltpu.VMEM((LATENT, NCHUNK, CHUNK), jnp.int32),
            pltpu.VMEM((LATENT, NCHUNK, CHUNK), jnp.int32),
            pltpu.VMEM((LATENT, B_PER_W), jnp.float32),
            pltpu.VMEM((LATENT, B_PER_W), jnp.float32),
            pltpu.VMEM((B_PER_W,), jnp.float32),
            pltpu.SemaphoreType.DMA,
            pltpu.SemaphoreType.DMA,
        ],
        compiler_params=pltpu.CompilerParams(needs_layout_passes=False),
    )
    return gather(users, items, uflat, iflat)


# speedup vs baseline: 44.0206x; 44.0206x over previous
import jax
import jax.numpy as jnp
from jax import lax
from jax.experimental import pallas as pl
from jax.experimental.pallas import tpu as pltpu
from jax.experimental.pallas import tpu_sc as plsc

NUM_CORES = 2
NUM_SUBCORES = 16
LANES = 16
NW = NUM_CORES * NUM_SUBCORES

NUM_ROWS = 1000000
BATCH = 16384
LATENT = 16
B_PER_W = BATCH // NW
CHUNK = 128
NCHUNK = B_PER_W // CHUNK
SUB = CHUNK // LANES

TPP = -(-NUM_ROWS // 128)
TILES_PER_W16 = -(-TPP // 16)
NTILE = 2 * TPP
DBLK_STRIDE = TPP * 1024
FLAT = 2 * DBLK_STRIDE
KRUN = 48
NRUN = -(-TILES_PER_W16 // KRUN)


def _retile_body(utab_ref, itab_ref, t3u_ref, t3i_ref,
                 tidx, stage0, stage1, semr0, semr1, semw0, semw1):
    wid = lax.axis_index("s") * NUM_CORES + lax.axis_index("c")
    tbl = wid % 2
    w16 = wid // 2

    start = jnp.minimum(w16 * TILES_PER_W16, TPP - TILES_PER_W16)

    stages = (stage0, stage1)
    semr = (semr0, semr1)
    semw = (semw0, semw1)
    iota = lax.iota(jnp.int32, LANES)

    def run(src_ref, dst_ref):
        step = 0
        for dblk in range(2):
            for u in range(NRUN):
                t0 = start + jnp.minimum(u * KRUN, TILES_PER_W16 - KRUN)
                for j in range(KRUN // LANES):
                    tidx[step, pl.ds(j * LANES, LANES)] = (
                        dblk * TPP + t0 + j * LANES + iota)
                step += 1

        writes = [None, None]
        step = 0
        for dblk in range(2):
            for u in range(NRUN):
                p = step % 2
                if writes[p] is not None:
                    writes[p].wait()
                t0 = start + jnp.minimum(u * KRUN, TILES_PER_W16 - KRUN)
                co = pl.multiple_of(t0 * 128, 128)

                def rd_loop(k, _):
                    cok = pl.multiple_of(co + k * 128, 128)
                    pltpu.make_async_copy(
                        src_ref.at[pl.ds(dblk * 8, 8), pl.ds(cok, 128)],
                        stages[p].at[k], semr[p]).start()
                    return 0

                lax.fori_loop(0, KRUN, rd_loop, 0)
                pltpu.make_async_copy(
                    dst_ref.at[pl.ds(0, KRUN)], stages[p], semr[p]).wait()
                wr = pltpu.make_async_copy(
                    stages[p], dst_ref.at[tidx.at[step]], semw[p])
                wr.start()
                writes[p] = wr
                step += 1
        for p in (0, 1):
            if writes[p] is not None:
                writes[p].wait()

    @pl.when(tbl == 0)
    def _():
        run(utab_ref, t3u_ref)

    @pl.when(tbl == 1)
    def _():
        run(itab_ref, t3i_ref)


def _gather_body(users_ref, items_ref, uflat_ref, iflat_ref, out_ref,
                 idx_u, idx_i, gidx_u, gidx_i, buf_u, buf_i, out_v,
                 sem_u, sem_i):
    wid = lax.axis_index("s") * NUM_CORES + lax.axis_index("c")
    base = wid * B_PER_W

    pltpu.sync_copy(users_ref.at[pl.ds(base, B_PER_W)], idx_u)
    pltpu.sync_copy(items_ref.at[pl.ds(base, B_PER_W)], idx_i)

    def build(c, _):
        for t in range(SUB):
            sl = pl.ds(c * CHUNK + t * LANES, LANES)
            tsl = pl.ds(t * LANES, LANES)
            ru = idx_u[sl]
            ri = idx_i[sl]
            bu = ((ru >> 7) << 10) + (ru & 127)
            bi = ((ri >> 7) << 10) + (ri & 127)
            for d in range(LATENT):
                cd = (d // 8) * DBLK_STRIDE + (d % 8) * 128
                gidx_u[d, c, tsl] = bu + cd
                gidx_i[d, c, tsl] = bi + cd
        return 0

    lax.fori_loop(0, NCHUNK, build, 0)

    copies = []
    for d in range(LATENT):
        for c in range(NCHUNK):
            sl = pl.ds(c * CHUNK, CHUNK)
            cu = pltpu.make_async_copy(
                uflat_ref.at[gidx_u.at[d].at[c]], buf_u.at[d].at[sl], sem_u)
            ci = pltpu.make_async_copy(
                iflat_ref.at[gidx_i.at[d].at[c]], buf_i.at[d].at[sl], sem_i)
            cu.start()
            ci.start()
            copies.append(cu)
            copies.append(ci)
    for cp in copies:
        cp.wait()

    def group(g, _):
        sl = pl.ds(g * LANES, LANES)
        acc = jnp.zeros((LANES,), jnp.float32)
        for d in range(LATENT):
            acc = acc + buf_u[d, sl] * buf_i[d, sl]
        out_v[sl] = 1.0 / (1.0 + jnp.exp(-acc))
        return 0

    lax.fori_loop(0, B_PER_W // LANES, group, 0)

    pltpu.sync_copy(out_v, out_ref.at[pl.ds(base, B_PER_W)])


@jax.jit
def kernel(users, items, user_table, item_table):
    ut2 = user_table.T
    it2 = item_table.T
    mesh = plsc.VectorSubcoreMesh(
        core_axis_name="c", subcore_axis_name="s",
        num_cores=NUM_CORES, num_subcores=NUM_SUBCORES)

    retile = pl.kernel(
        _retile_body,
        out_type=(
            jax.ShapeDtypeStruct((NTILE, 8, 128), jnp.float32),
            jax.ShapeDtypeStruct((NTILE, 8, 128), jnp.float32),
        ),
        mesh=mesh,
        scratch_types=[
            pltpu.VMEM((2 * NRUN, KRUN), jnp.int32),
            pltpu.VMEM((KRUN, 8, 128), jnp.float32),
            pltpu.VMEM((KRUN, 8, 128), jnp.float32),
            pltpu.SemaphoreType.DMA,
            pltpu.SemaphoreType.DMA,
            pltpu.SemaphoreType.DMA,
            pltpu.SemaphoreType.DMA,
        ],
    )
    t3u, t3i = retile(ut2, it2)
    uflat = t3u.reshape(FLAT)
    iflat = t3i.reshape(FLAT)

    gather = pl.kernel(
        _gather_body,
        out_type=jax.ShapeDtypeStruct((BATCH,), jnp.float32),
        mesh=mesh,
        scratch_types=[
            pltpu.VMEM((B_PER_W,), jnp.int32),
            pltpu.VMEM((B_PER_W,), jnp.int32),
            pltpu.VMEM((LATENT, NCHUNK, CHUNK), jnp.int32),
            pltpu.VMEM((LATENT, NCHUNK, CHUNK), jnp.int32),
            pltpu.VMEM((LATENT, B_PER_W), jnp.float32),
            pltpu.VMEM((LATENT, B_PER_W), jnp.float32),
            pltpu.VMEM((B_PER_W,), jnp.float32),
            pltpu.SemaphoreType.DMA,
            pltpu.SemaphoreType.DMA,
        ],
        compiler_params=pltpu.CompilerParams(needs_layout_passes=False),
    )
    return gather(users, items, uflat, iflat)
